# Initial kernel scaffold; baseline (speedup 1.0000x reference)
import jax
import jax.numpy as jnp
from jax.experimental import pallas as pl


def kernel(arm_locs, arm_scores, odm_locs, odm_scores, boxes, labels, priors_cxcy):
    raise NotImplementedError("write your pallas kernel here")


# speedup vs baseline: 29.4489x; 29.4489x over previous
import functools

import jax
import jax.numpy as jnp
from jax import lax
from jax.experimental import pallas as pl
from jax.experimental.pallas import tpu as pltpu

_B, _P, _O, _C = 16, 8732, 12, 21
_THR, _RATIO, _THETA, _ALPHA = 0.5, 3, 0.01, 1.0
_SL, _LN = 8, 1152
_PP = _SL * _LN


def _diou(px0, py0, px1, py1, tx0, ty0, tx1, ty1):
    eps = 1e-7
    ix0 = jnp.maximum(px0, tx0)
    iy0 = jnp.maximum(py0, ty0)
    ix1 = jnp.minimum(px1, tx1)
    iy1 = jnp.minimum(py1, ty1)
    inter = jnp.clip(ix1 - ix0, 0.0, None) * jnp.clip(iy1 - iy0, 0.0, None)
    ap = jnp.clip(px1 - px0, 0.0, None) * jnp.clip(py1 - py0, 0.0, None)
    at = jnp.clip(tx1 - tx0, 0.0, None) * jnp.clip(ty1 - ty0, 0.0, None)
    union = ap + at - inter
    iou = inter / (union + eps)
    cpx = (px0 + px1) / 2
    cpy = (py0 + py1) / 2
    ctx = (tx0 + tx1) / 2
    cty = (ty0 + ty1) / 2
    rho2 = (cpx - ctx) ** 2 + (cpy - cty) ** 2
    ex0 = jnp.minimum(px0, tx0)
    ey0 = jnp.minimum(py0, ty0)
    ex1 = jnp.maximum(px1, tx1)
    ey1 = jnp.maximum(py1, ty1)
    c2 = (ex1 - ex0) ** 2 + (ey1 - ey0) ** 2 + eps
    return 1.0 - (iou - rho2 / c2)


def _stage1_body(al_ref, as_ref, ol_ref, os_ref, pr_ref, bx_ref, lb_ref,
                 conf_a_ref, conf_o_ref, part_ref):
    shp = (_SL, _LN)
    row = lax.broadcasted_iota(jnp.int32, shp, 0)
    coli = lax.broadcasted_iota(jnp.int32, shp, 1)
    pidx = row * _LN + coli
    pad = pidx >= _P

    pcx, pcy, pw, ph = pr_ref[0], pr_ref[1], pr_ref[2], pr_ref[3]
    px0 = pcx - pw / 2
    py0 = pcy - ph / 2
    px1 = pcx + pw / 2
    py1 = pcy + ph / 2

    g0, g1, g2, g3 = al_ref[0, 0], al_ref[0, 1], al_ref[0, 2], al_ref[0, 3]
    acx = g0 * pw / 10 + pcx
    acy = g1 * ph / 10 + pcy
    aw = jnp.exp(g2 / 5) * pw
    ah = jnp.exp(g3 / 5) * ph
    ax0 = acx - aw / 2
    ay0 = acy - ah / 2
    ax1 = acx + aw / 2
    ay1 = acy + ah / 2

    boxes = [[bx_ref[0, o, j] for j in range(4)] for o in range(_O)]
    labels = [lb_ref[0, 0, o] for o in range(_O)]

    def run_match(x0, y0, x1, y1, thr):
        area2 = (x1 - x0) * (y1 - y0)
        best = jnp.full(shp, -1.0, jnp.float32)
        obj = jnp.zeros(shp, jnp.int32)
        mx_l, pf_l = [], []
        for o in range(_O):
            bx0, by0, bx1, by1 = boxes[o]
            a1 = (bx1 - bx0) * (by1 - by0)
            inter = (jnp.clip(jnp.minimum(bx1, x1) - jnp.maximum(bx0, x0), 0.0, None)
                     * jnp.clip(jnp.minimum(by1, y1) - jnp.maximum(by0, y0), 0.0, None))
            ov = inter / jnp.maximum(a1 + area2 - inter, 1e-10)
            upd = ov > best
            best = jnp.where(upd, ov, best)
            obj = jnp.where(upd, o, obj)
            mx = jnp.max(ov)
            pf = jnp.min(jnp.where(ov == mx, pidx, _PP))
            mx_l.append(mx)
            pf_l.append(pf)
        obj0 = obj
        ofp = best
        for o in range(_O):
            m = mx_l[o] > 0.0
            hit = pidx == pf_l[o]
            ofp = jnp.where(hit & m, jnp.maximum(ofp, 1.0), ofp)
            obj = jnp.where(hit & m, o, jnp.where(hit & (~m), obj0, obj))
        lfp = jnp.zeros(shp, jnp.int32)
        tx0 = jnp.zeros(shp, jnp.float32)
        ty0 = jnp.zeros(shp, jnp.float32)
        tx1 = jnp.zeros(shp, jnp.float32)
        ty1 = jnp.zeros(shp, jnp.float32)
        for o in range(_O):
            sel = obj == o
            lfp = jnp.where(sel, labels[o], lfp)
            tx0 = jnp.where(sel, boxes[o][0], tx0)
            ty0 = jnp.where(sel, boxes[o][1], ty0)
            tx1 = jnp.where(sel, boxes[o][2], tx1)
            ty1 = jnp.where(sel, boxes[o][3], ty1)
        lfp = jnp.where(ofp < thr, 0, lfp)
        return lfp, (tx0, ty0, tx1, ty1)

    lfp_a, tla = run_match(px0, py0, px1, py1, _THR - 0.2)
    pos_a = lfp_a > 0
    n_pos_a = jnp.sum(jnp.where(pos_a, 1.0, 0.0))
    d_a = _diou(ax0, ay0, ax1, ay1, *tla)
    dsum_a = jnp.sum(jnp.where(pos_a, d_a, 0.0))
    s0, s1 = as_ref[0, 0], as_ref[0, 1]
    mx2 = jnp.maximum(s0, s1)
    lse2 = mx2 + jnp.log(jnp.exp(s0 - mx2) + jnp.exp(s1 - mx2))
    ce_a = lse2 - jnp.where(pos_a, s1, s0)
    pos_ce_a = jnp.sum(jnp.where(pos_a, ce_a, 0.0))
    conf_a_ref[0] = jnp.where(pos_a | pad, 0.0, ce_a)

    lfp_o, tlo = run_match(ax0, ay0, ax1, ay1, _THR)
    easy = jnp.exp(s1 - lse2) < _THETA
    pos_o = (lfp_o > 0) & (~easy)
    n_pos_o = jnp.sum(jnp.where(pos_o, 1.0, 0.0))

    acx2 = (ax0 + ax1) / 2
    acy2 = (ay0 + ay1) / 2
    aw2 = ax1 - ax0
    ah2 = ay1 - ay0
    h0, h1, h2, h3 = ol_ref[0, 0], ol_ref[0, 1], ol_ref[0, 2], ol_ref[0, 3]
    ocx = h0 * aw2 / 10 + acx2
    ocy = h1 * ah2 / 10 + acy2
    ow = jnp.exp(h2 / 5) * aw2
    oh = jnp.exp(h3 / 5) * ah2
    ox0 = ocx - ow / 2
    oy0 = ocy - oh / 2
    ox1 = ocx + ow / 2
    oy1 = ocy + oh / 2
    d_o = _diou(ox0, oy0, ox1, oy1, *tlo)
    dsum_o = jnp.sum(jnp.where(pos_o, d_o, 0.0))

    cls_o = lfp_o
    logits = [os_ref[0, c] for c in range(_C)]
    mxc = logits[0]
    for c in range(1, _C):
        mxc = jnp.maximum(mxc, logits[c])
    sume = jnp.exp(logits[0] - mxc)
    for c in range(1, _C):
        sume = sume + jnp.exp(logits[c] - mxc)
    lsec = mxc + jnp.log(sume)
    chosen = jnp.zeros(shp, jnp.float32)
    for c in range(_C):
        chosen = jnp.where(cls_o == c, logits[c], chosen)
    ce_o = lsec - chosen
    pos_ce_o = jnp.sum(jnp.where(pos_o, ce_o, 0.0))
    conf_o_ref[0] = jnp.where(pos_o | easy | pad, 0.0, ce_o)

    li = lax.broadcasted_iota(jnp.int32, (1, 128), 1)
    vals = [n_pos_a, pos_ce_a, dsum_a, n_pos_o, pos_ce_o, dsum_o]
    acc = jnp.zeros((1, 128), jnp.float32)
    for j, v in enumerate(vals):
        acc = jnp.where(li == j, v, acc)
    part_ref[0] = acc


def _mine_body(conf_a_ref, conf_o_ref, part_ref, out_ref):
    pa = part_ref[:, 0, :]
    li = lax.broadcasted_iota(jnp.int32, (_B, 128), 1)

    def col(j):
        return jnp.sum(jnp.where(li == j, pa, 0.0), axis=1, keepdims=True)

    npa, cepa, da = col(0), col(1), col(2)
    npo, cepo, do_ = col(3), col(4), col(5)

    def neg_sum(v, npos):
        k = (_RATIO * npos).astype(jnp.int32)
        rmax = jnp.max(v, axis=1, keepdims=True)
        hi = lax.bitcast_convert_type(rmax, jnp.int32) + 1
        lo = jnp.zeros_like(hi)

        def body(_, c):
            lo, hi = c
            mid = lo + (hi - lo) // 2
            t = lax.bitcast_convert_type(mid, jnp.float32)
            cnt = jnp.sum((v > t).astype(jnp.int32), axis=1, keepdims=True)
            p = cnt < k
            return jnp.where(p, lo, mid + 1), jnp.where(p, mid, hi)

        lo, hi = lax.fori_loop(0, 31, body, (lo, hi))
        t = lax.bitcast_convert_type(lo, jnp.float32)
        cnt = jnp.sum((v > t).astype(jnp.int32), axis=1, keepdims=True)
        sab = jnp.sum(jnp.where(v > t, v, 0.0), axis=1, keepdims=True)
        neg = sab + (k - cnt).astype(jnp.float32) * t
        return jnp.sum(neg)

    neg_a = neg_sum(conf_a_ref[...], npa)
    neg_o = neg_sum(conf_o_ref[...], npo)

    npa_t = jnp.sum(npa)
    npo_t = jnp.sum(npo)
    conf_a = (neg_a + jnp.sum(cepa)) / npa_t
    loc_a = jnp.sum(da) / jnp.maximum(npa_t, 1.0)
    conf_o = (neg_o + jnp.sum(cepo)) / npo_t
    loc_o = jnp.sum(do_) / jnp.maximum(npo_t, 1.0)
    out_ref[0, 0] = conf_a + _ALPHA * loc_a + conf_o + _ALPHA * loc_o


def _prep(x):
    b, p, k = x.shape
    xt = jnp.swapaxes(x, 1, 2)
    xt = jnp.pad(xt, ((0, 0), (0, 0), (0, _PP - p)))
    return xt.reshape(b, k, _SL, _LN)


@jax.jit
def kernel(arm_locs, arm_scores, odm_locs, odm_scores, boxes, labels, priors_cxcy):
    al = _prep(arm_locs)
    asr = _prep(arm_scores)
    ol = _prep(odm_locs)
    osr = _prep(odm_scores)
    pr = jnp.pad(jnp.swapaxes(priors_cxcy, 0, 1),
                 ((0, 0), (0, _PP - _P))).reshape(4, _SL, _LN)
    labels = labels.astype(jnp.int32).reshape(_B, 1, _O)

    conf_a, conf_o, part = pl.pallas_call(
        _stage1_body,
        grid=(_B,),
        in_specs=[
            pl.BlockSpec((1, 4, _SL, _LN), lambda b: (b, 0, 0, 0)),
            pl.BlockSpec((1, 2, _SL, _LN), lambda b: (b, 0, 0, 0)),
            pl.BlockSpec((1, 4, _SL, _LN), lambda b: (b, 0, 0, 0)),
            pl.BlockSpec((1, _C, _SL, _LN), lambda b: (b, 0, 0, 0)),
            pl.BlockSpec((4, _SL, _LN), lambda b: (0, 0, 0)),
            pl.BlockSpec((1, _O, 4), lambda b: (b, 0, 0), memory_space=pltpu.SMEM),
            pl.BlockSpec((1, 1, _O), lambda b: (b, 0, 0), memory_space=pltpu.SMEM),
        ],
        out_specs=[
            pl.BlockSpec((1, _SL, _LN), lambda b: (b, 0, 0)),
            pl.BlockSpec((1, _SL, _LN), lambda b: (b, 0, 0)),
            pl.BlockSpec((1, 1, 128), lambda b: (b, 0, 0)),
        ],
        out_shape=[
            jax.ShapeDtypeStruct((_B, _SL, _LN), jnp.float32),
            jax.ShapeDtypeStruct((_B, _SL, _LN), jnp.float32),
            jax.ShapeDtypeStruct((_B, 1, 128), jnp.float32),
        ],
    )(al, asr, ol, osr, pr, boxes, labels)

    out = pl.pallas_call(
        _mine_body,
        in_specs=[
            pl.BlockSpec((_B, _PP), lambda: (0, 0)),
            pl.BlockSpec((_B, _PP), lambda: (0, 0)),
            pl.BlockSpec((_B, 1, 128), lambda: (0, 0, 0)),
        ],
        out_specs=pl.BlockSpec((1, 1), lambda: (0, 0), memory_space=pltpu.SMEM),
        out_shape=jax.ShapeDtypeStruct((1, 1), jnp.float32),
    )(conf_a.reshape(_B, _PP), conf_o.reshape(_B, _PP), part)
    return out.reshape(())
